# baseline (speedup 1.0000x reference)
import functools

import jax
import jax.numpy as jnp
from jax import lax
from jax.experimental import pallas as pl
from jax.experimental.pallas import tpu as pltpu
from jax.experimental.pallas import tpu_sc as plsc

L = 16
NC, NS = 2, 16
NW = NC * NS
A_IN = 60
A_OUT = 60
K = 6
NB = 4
ROW_UNROLL = 4


def _anchor_prop_sc(rows_total, chunk_rows):
    rows_per_w = rows_total // NW
    n_chunks = rows_per_w // chunk_rows
    in_words = chunk_rows * A_IN
    out_words = chunk_rows * A_OUT
    mesh = plsc.VectorSubcoreMesh(core_axis_name="c", subcore_axis_name="s",
                                  num_cores=NC, num_subcores=NS)

    @functools.partial(
        pl.kernel,
        out_type=jax.ShapeDtypeStruct((rows_total * A_OUT,), jnp.float32),
        mesh=mesh,
        scratch_types=[
            pltpu.VMEM((in_words,), jnp.float32),
            pltpu.VMEM((out_words,), jnp.float32),
            pltpu.VMEM((K * NB * L,), jnp.int32),
            pltpu.VMEM((K * NB * L,), jnp.float32),
        ],
        compiler_params=pltpu.CompilerParams(needs_layout_passes=False),
    )
    def k(feats_hbm, idxt_hbm, wt_hbm, out_hbm, in_v, out_v, idx_v, w_v):
        wid = lax.axis_index("s") * NC + lax.axis_index("c")
        base_w = wid * rows_per_w
        pltpu.sync_copy(idxt_hbm, idx_v)
        pltpu.sync_copy(wt_hbm, w_v)

        lane = lax.iota(jnp.int32, L)

        def do_chunk(c, _):
            row0 = base_w + c * chunk_rows
            pltpu.sync_copy(feats_hbm.at[pl.ds(row0 * A_IN, in_words)], in_v)

            for b in range(NB):
                idx_b = [idx_v[pl.ds((j * NB + b) * L, L)] for j in range(K)]
                w_b = [w_v[pl.ds((j * NB + b) * L, L)] for j in range(K)]
                out_cols = lane + b * L
                out_mask = out_cols < A_OUT

                def do_rows(i, _, idx_b=idx_b, w_b=w_b, out_cols=out_cols,
                            out_mask=out_mask):
                    for u in range(ROW_UNROLL):
                        r = i * ROW_UNROLL + u
                        acc = w_b[0] * plsc.load_gather(
                            in_v, [idx_b[0] + r * A_IN])
                        for j in range(1, K):
                            acc = acc + w_b[j] * plsc.load_gather(
                                in_v, [idx_b[j] + r * A_IN])
                        plsc.store_scatter(out_v, [out_cols + r * A_OUT], acc,
                                           mask=out_mask)
                    return ()

                lax.fori_loop(0, chunk_rows // ROW_UNROLL, do_rows, (),
                              unroll=1)

            pltpu.sync_copy(out_v,
                            out_hbm.at[pl.ds(row0 * A_OUT, out_words)])
            return ()

        lax.fori_loop(0, n_chunks, do_chunk, (), unroll=1)

    return k


@jax.jit
def kernel(xyz, feats, idx, w, anchor_out):
    B, C, N, A = feats.shape
    rows = B * C * N
    feats2 = feats.reshape(rows * A)
    idx_t = jnp.zeros((K, NB * L), jnp.int32).at[:, :A_OUT].set(
        idx.astype(jnp.int32).T)
    w_t = jnp.zeros((K, NB * L), jnp.float32).at[:, :A_OUT].set(w.T)
    out2 = _anchor_prop_sc(rows, 256)(feats2, idx_t.reshape(-1),
                                      w_t.reshape(-1))
    return (xyz, out2.reshape(B, C, N, A_OUT), anchor_out)


# speedup vs baseline: 3.4148x; 3.4148x over previous
import functools

import jax
import jax.numpy as jnp
from jax import lax
from jax.experimental import pallas as pl
from jax.experimental.pallas import tpu as pltpu
from jax.experimental.pallas import tpu_sc as plsc

L = 16
NC, NS = 2, 16
A_IN = 60
A_OUT = 60
K = 6
NB = 4
M_WORDS = A_IN * A_OUT


def _build_mt_sc():
    mesh = plsc.VectorSubcoreMesh(core_axis_name="c", subcore_axis_name="s",
                                  num_cores=NC, num_subcores=NS)

    @functools.partial(
        pl.kernel,
        out_type=jax.ShapeDtypeStruct((M_WORDS,), jnp.float32),
        mesh=mesh,
        scratch_types=[
            pltpu.VMEM((M_WORDS,), jnp.float32),
            pltpu.VMEM((K * NB * L,), jnp.int32),
            pltpu.VMEM((K * NB * L,), jnp.float32),
        ],
        compiler_params=pltpu.CompilerParams(needs_layout_passes=False),
    )
    def k(idxt_hbm, wt_hbm, mt_hbm, m_v, idx_v, w_v):
        wid = lax.axis_index("s") * NC + lax.axis_index("c")

        @pl.when(wid == 0)
        def _():
            pltpu.sync_copy(idxt_hbm, idx_v)
            pltpu.sync_copy(wt_hbm, w_v)
            zero = jnp.zeros((L,), jnp.float32)

            def zero_body(i, _):
                m_v[pl.ds(i * L, L)] = zero
                return ()

            lax.fori_loop(0, M_WORDS // L, zero_body, (), unroll=4)

            lane = lax.iota(jnp.int32, L)
            for b in range(NB):
                ao = lane + b * L
                mask = ao < A_OUT
                for j in range(K):
                    a_j = idx_v[pl.ds((j * NB + b) * L, L)]
                    w_j = w_v[pl.ds((j * NB + b) * L, L)]
                    plsc.store_scatter(m_v, [a_j * A_OUT + ao], w_j,
                                       mask=mask)
            pltpu.sync_copy(m_v, mt_hbm)

    return k


def _matmul_tc(rows, blk):
    grid = rows // blk

    def body(x_ref, m_ref, o_ref):
        o_ref[...] = jnp.dot(x_ref[...], m_ref[...],
                             preferred_element_type=jnp.float32)

    return pl.pallas_call(
        body,
        grid=(grid,),
        in_specs=[
            pl.BlockSpec((blk, A_IN), lambda i: (i, 0)),
            pl.BlockSpec((A_IN, A_OUT), lambda i: (0, 0)),
        ],
        out_specs=pl.BlockSpec((blk, A_OUT), lambda i: (i, 0)),
        out_shape=jax.ShapeDtypeStruct((rows, A_OUT), jnp.float32),
    )


@jax.jit
def kernel(xyz, feats, idx, w, anchor_out):
    B, C, N, A = feats.shape
    rows = B * C * N
    feats2 = feats.reshape(rows, A)
    idx_t = jnp.zeros((K, NB * L), jnp.int32).at[:, :A_OUT].set(
        idx.astype(jnp.int32).T)
    w_t = jnp.zeros((K, NB * L), jnp.float32).at[:, :A_OUT].set(w.T)
    mt = _build_mt_sc()(idx_t.reshape(-1), w_t.reshape(-1))
    out2 = _matmul_tc(rows, 2048)(feats2, mt.reshape(A_IN, A_OUT))
    return (xyz, out2.reshape(B, C, N, A_OUT), anchor_out)
